# Initial kernel scaffold; baseline (speedup 1.0000x reference)
import jax
import jax.numpy as jnp
from jax.experimental import pallas as pl


def kernel(diner_idx, reviewer_idx, edge_index, diner_table, reviewer_table, W1, b1, W2, b2, fc_W, fc_b):
    raise NotImplementedError("write your pallas kernel here")


# speedup vs baseline: 1.8909x; 1.8909x over previous
import functools

import jax
import jax.numpy as jnp
from jax import lax
from jax.experimental import pallas as pl
from jax.experimental.pallas import tpu as pltpu
from jax.experimental.pallas import tpu_sc as plsc

_L = 16
_NC = 2
_NS = 16
_NW = _NC * _NS
_H = 32
_TC_BLK = 2000


def _tc_matvec(d_tab, r_tab, w_d, w_r, bias):
  rows = d_tab.shape[0]
  grid = pl.cdiv(rows, _TC_BLK)

  def body(d_ref, r_ref, wd_ref, wr_ref, b_ref, u_ref, v_ref):
    u_ref[...] = jnp.dot(d_ref[...], wd_ref[...],
                         preferred_element_type=jnp.float32) + b_ref[0, 0]
    v_ref[...] = jnp.dot(r_ref[...], wr_ref[...],
                         preferred_element_type=jnp.float32)

  return pl.pallas_call(
      body,
      grid=(grid,),
      in_specs=[
          pl.BlockSpec((_TC_BLK, _H), lambda i: (i, 0)),
          pl.BlockSpec((_TC_BLK, _H), lambda i: (i, 0)),
          pl.BlockSpec((_H, 1), lambda i: (0, 0)),
          pl.BlockSpec((_H, 1), lambda i: (0, 0)),
          pl.BlockSpec((1, 1), lambda i: (0, 0), memory_space=pltpu.SMEM),
      ],
      out_specs=[
          pl.BlockSpec((_TC_BLK, 1), lambda i: (i, 0)),
          pl.BlockSpec((_TC_BLK, 1), lambda i: (i, 0)),
      ],
      out_shape=[
          jax.ShapeDtypeStruct((rows, 1), jnp.float32),
          jax.ShapeDtypeStruct((rows, 1), jnp.float32),
      ],
  )(d_tab, r_tab, w_d, w_r, bias)


def _build_sc_gather(n_rows: int, b_per_w: int):
  mesh = plsc.VectorSubcoreMesh(core_axis_name="c", subcore_axis_name="s")
  b_pad = b_per_w * _NW

  @functools.partial(
      pl.kernel,
      out_type=jax.ShapeDtypeStruct((b_pad,), jnp.float32),
      mesh=mesh,
      compiler_params=pltpu.CompilerParams(needs_layout_passes=False),
      scratch_types=[
          pltpu.VMEM((n_rows,), jnp.float32),
          pltpu.VMEM((n_rows,), jnp.float32),
          pltpu.VMEM((b_per_w,), jnp.int32),
          pltpu.VMEM((b_per_w,), jnp.int32),
          pltpu.VMEM((b_per_w,), jnp.float32),
          pltpu.SemaphoreType.DMA,
          pltpu.SemaphoreType.DMA,
      ],
  )
  def sc_kernel(u_hbm, v_hbm, d_idx_hbm, r_idx_hbm, out_hbm,
                u_v, v_v, d_idx_v, r_idx_v, out_v, usem, vsem):
    wid = lax.axis_index("s") * _NC + lax.axis_index("c")
    base = wid * b_per_w
    ucp = pltpu.async_copy(u_hbm, u_v, usem)
    vcp = pltpu.async_copy(v_hbm, v_v, vsem)
    pltpu.sync_copy(d_idx_hbm.at[pl.ds(base, b_per_w)], d_idx_v)
    pltpu.sync_copy(r_idx_hbm.at[pl.ds(base, b_per_w)], r_idx_v)
    ucp.wait()
    vcp.wait()

    def body(g, carry):
      sl = pl.ds(g * _L, _L)
      out_v[sl] = (plsc.load_gather(u_v, [d_idx_v[sl]])
                   + plsc.load_gather(v_v, [r_idx_v[sl]]))
      return carry

    lax.fori_loop(0, b_per_w // _L, body, 0)
    pltpu.sync_copy(out_v, out_hbm.at[pl.ds(base, b_per_w)])

  return sc_kernel


def kernel(diner_idx, reviewer_idx, edge_index, diner_table, reviewer_table,
           W1, b1, W2, b2, fc_W, fc_b):
  del edge_index, W1, b1, W2, b2
  b = diner_idx.shape[0]
  n_rows = diner_table.shape[0]
  fc_W = fc_W.astype(jnp.float32)
  u2d, v2d = _tc_matvec(diner_table.astype(jnp.float32),
                        reviewer_table.astype(jnp.float32),
                        fc_W[:_H], fc_W[_H:],
                        fc_b.astype(jnp.float32).reshape(1, 1))
  u = u2d.reshape(n_rows)
  v = v2d.reshape(n_rows)

  grp = _NW * _L
  b_per_w = (-(-b // grp)) * _L
  pad = b_per_w * _NW - b
  d_idx = jnp.pad(diner_idx.astype(jnp.int32), (0, pad))
  r_idx = jnp.pad(reviewer_idx.astype(jnp.int32), (0, pad))
  out = _build_sc_gather(n_rows, b_per_w)(u, v, d_idx, r_idx)
  return out[:b]
